# baseline (speedup 1.0000x reference)
import functools

import jax
import jax.numpy as jnp
from jax import lax
from jax.experimental import pallas as pl
from jax.experimental.pallas import tpu as pltpu
from jax.experimental.pallas import tpu_sc as plsc

N = 10000
E = 320000
D = 128
LAYERS = 4
EPS = 1e-05
LN_EPS = 1e-05

NC = 2
NS = 16
NW = NC * NS
CHUNK = 128
SB = 8
NSB = 10
NCH = SB * NSB
EP = NW * NCH * CHUNK
NPAD = 10240
RPT = NPAD // NS

_mesh = plsc.VectorSubcoreMesh(core_axis_name="c", subcore_axis_name="s")


@functools.partial(
    pl.kernel,
    out_type=(
        jax.ShapeDtypeStruct((NPAD,), jnp.float32),
        jax.ShapeDtypeStruct((NPAD,), jnp.float32),
    ),
    mesh=_mesh,
    scratch_types=[
        pltpu.VMEM((NSB, SB, CHUNK), jnp.int32),
        pltpu.VMEM((CHUNK,), jnp.float32),
        pltpu.VMEM_SHARED((NPAD,), jnp.float32),
    ],
)
def _deg_kernel(dst_hbm, zeros1d_hbm, ones_hbm, d0_hbm, d1_hbm,
                idxd_v, ones_v, deg_sp):
    cid = lax.axis_index("c")
    sid = lax.axis_index("s")
    w = cid * NS + sid
    pltpu.sync_copy(zeros1d_hbm, deg_sp.at[pl.ds(sid * RPT, RPT)])
    pltpu.sync_copy(ones_hbm, ones_v)
    pltpu.sync_copy(dst_hbm.at[w], idxd_v)
    plsc.subcore_barrier()

    @pl.loop(0, NSB)
    def _(sb):
        for k in range(SB):
            pltpu.sync_copy(ones_v, deg_sp.at[idxd_v.at[sb, k]], add=True)

    plsc.subcore_barrier()

    @pl.when(jnp.logical_and(sid == 0, cid == 0))
    def _():
        pltpu.sync_copy(deg_sp, d0_hbm)

    @pl.when(jnp.logical_and(sid == 0, cid == 1))
    def _():
        pltpu.sync_copy(deg_sp, d1_hbm)


@functools.partial(
    pl.kernel,
    out_type=(
        jax.ShapeDtypeStruct((NPAD, D), jnp.float32),
        jax.ShapeDtypeStruct((NPAD, D), jnp.float32),
    ),
    mesh=_mesh,
    scratch_types=[
        pltpu.VMEM((2, SB, CHUNK), jnp.int32),
        pltpu.VMEM((2, SB, CHUNK), jnp.int32),
        pltpu.VMEM((CHUNK, D), jnp.float32),
        pltpu.VMEM_SHARED((NPAD, D), jnp.float32),
        pltpu.SemaphoreType.DMA,
        pltpu.SemaphoreType.DMA,
    ],
)
def _agg_kernel(y_hbm, src_hbm, dst_hbm, z0_hbm, z1_hbm,
                idxs_v, idxd_v, rows_v, z_sp, sem, semi):
    cid = lax.axis_index("c")
    sid = lax.axis_index("s")
    w = cid * NS + sid

    pltpu.sync_copy(src_hbm.at[w, 0], idxs_v.at[0])
    pltpu.sync_copy(dst_hbm.at[w, 0], idxd_v.at[0])

    zv = jnp.zeros((16,), jnp.float32)

    @pl.loop(0, CHUNK)
    def _(i):
        for jj in range(D // 16):
            rows_v[i, pl.ds(jj * 16, 16)] = zv

    @pl.loop(0, RPT // CHUNK)
    def _(r):
        pltpu.sync_copy(rows_v, z_sp.at[pl.ds(sid * RPT + r * CHUNK, CHUNK)])

    plsc.subcore_barrier()

    @pl.loop(0, NSB, step=2)
    def _(sb):
        for t in range(2):
            cur = sb + t
            nxt = 1 - t

            @pl.when(cur + 1 < NSB)
            def _():
                pltpu.async_copy(src_hbm.at[w, cur + 1], idxs_v.at[nxt],
                                 semi)
                pltpu.async_copy(dst_hbm.at[w, cur + 1], idxd_v.at[nxt],
                                 semi)

            for k in range(SB):
                pltpu.async_copy(y_hbm.at[idxs_v.at[t, k]], rows_v,
                                 sem).wait()
                pltpu.sync_copy(rows_v, z_sp.at[idxd_v.at[t, k]], add=True)

            @pl.when(cur + 1 < NSB)
            def _():
                pltpu.make_async_copy(src_hbm.at[w, 0], idxs_v.at[nxt],
                                      semi).wait()
                pltpu.make_async_copy(dst_hbm.at[w, 0], idxd_v.at[nxt],
                                      semi).wait()

    plsc.subcore_barrier()

    @pl.when(cid == 0)
    def _():
        pltpu.sync_copy(z_sp.at[pl.ds(sid * RPT, RPT)],
                        z0_hbm.at[pl.ds(sid * RPT, RPT)])

    @pl.when(cid == 1)
    def _():
        pltpu.sync_copy(z_sp.at[pl.ds(sid * RPT, RPT)],
                        z1_hbm.at[pl.ds(sid * RPT, RPT)])


def _prep_body(degs_ref, x_ref, w_ref, y_ref, dinv_ref):
    d = degs_ref[:, 0] + degs_ref[:, 1] + 1.0
    dinv = lax.rsqrt(d)[:, None]
    dinv_ref[...] = jnp.broadcast_to(dinv, x_ref.shape)
    y_ref[...] = dinv * jnp.dot(x_ref[...], w_ref[...],
                                preferred_element_type=jnp.float32)


def _post_body(layer, z0_ref, z1_ref, y_ref, xin_ref, dinv_ref,
               b_ref, g_ref, bt_ref, wn_ref, h_ref, yn_ref):
    dinv = dinv_ref[...]
    h = dinv * (z0_ref[...] + z1_ref[...] + y_ref[...]) + b_ref[...]
    h = jnp.where(jnp.isnan(h), jnp.float32(0.0), h)
    h = jnp.where(jnp.isinf(h) & (h > 0), jnp.float32(EPS), h)
    h = jnp.where(jnp.isinf(h) & (h < 0), jnp.float32(-EPS), h)
    mu = jnp.mean(h, axis=-1, keepdims=True)
    var = jnp.mean((h - mu) ** 2, axis=-1, keepdims=True)
    h = (h - mu) / jnp.sqrt(var + LN_EPS) * g_ref[...] + bt_ref[...]
    if layer > 0:
        h = h + xin_ref[...]
    if layer < LAYERS - 1:
        h = jax.nn.relu(h)
    h_ref[...] = h
    if layer < LAYERS - 1:
        yn_ref[...] = dinv * jnp.dot(h, wn_ref[...],
                                     preferred_element_type=jnp.float32)


_BN = 1000


def _tc_prep(degs, x, w0):
    return pl.pallas_call(
        _prep_body,
        grid=(N // _BN,),
        in_specs=[
            pl.BlockSpec((_BN, 2), lambda i: (i, 0)),
            pl.BlockSpec((_BN, D), lambda i: (i, 0)),
            pl.BlockSpec((D, D), lambda i: (0, 0)),
        ],
        out_specs=[
            pl.BlockSpec((_BN, D), lambda i: (i, 0)),
            pl.BlockSpec((_BN, D), lambda i: (i, 0)),
        ],
        out_shape=[
            jax.ShapeDtypeStruct((N, D), jnp.float32),
            jax.ShapeDtypeStruct((N, D), jnp.float32),
        ],
    )(degs, x, w0)


def _tc_post(layer, z0, z1, y, xin, dinv2d, bl, gl, btl, wn):
    last = layer == LAYERS - 1
    if last:
        def body(z0r, z1r, yr, xr, dr, br, gr, btr, wr, hr):
            _post_body(layer, z0r, z1r, yr, xr, dr, br, gr, btr, wr, hr, None)
        out_specs = [pl.BlockSpec((_BN, D), lambda i: (i, 0))]
        out_shape = [jax.ShapeDtypeStruct((N, D), jnp.float32)]
    else:
        body = functools.partial(_post_body, layer)
        out_specs = [pl.BlockSpec((_BN, D), lambda i: (i, 0))] * 2
        out_shape = [jax.ShapeDtypeStruct((N, D), jnp.float32)] * 2
    res = pl.pallas_call(
        body,
        grid=(N // _BN,),
        in_specs=[
            pl.BlockSpec((_BN, D), lambda i: (i, 0)),
            pl.BlockSpec((_BN, D), lambda i: (i, 0)),
            pl.BlockSpec((_BN, D), lambda i: (i, 0)),
            pl.BlockSpec((_BN, D), lambda i: (i, 0)),
            pl.BlockSpec((_BN, D), lambda i: (i, 0)),
            pl.BlockSpec((1, D), lambda i: (0, 0)),
            pl.BlockSpec((1, D), lambda i: (0, 0)),
            pl.BlockSpec((1, D), lambda i: (0, 0)),
            pl.BlockSpec((D, D), lambda i: (0, 0)),
        ],
        out_specs=out_specs,
        out_shape=out_shape,
    )(z0, z1, y, xin, dinv2d, bl, gl, btl, wn)
    return res if not last else (res[0], None)


@jax.jit
def kernel(x, edge_index, W, b, gamma, beta):
    npad_e = EP - E
    src_r = jnp.concatenate(
        [edge_index[0], jnp.zeros((npad_e,), jnp.int32)]
    ).reshape(NW, NSB, SB, CHUNK)
    dst_r = jnp.concatenate(
        [edge_index[1], jnp.full((npad_e,), NPAD - 1, jnp.int32)]
    ).reshape(NW, NSB, SB, CHUNK)
    zeros1d = jnp.zeros((RPT,), jnp.float32)
    ones_c = jnp.ones((CHUNK,), jnp.float32)

    d0, d1 = _deg_kernel(dst_r, zeros1d, ones_c)
    degs = jnp.stack([d0[:N], d1[:N]], axis=1)
    y, dinv2d = _tc_prep(degs, x, W[0])

    h = x
    for l in range(LAYERS):
        z0, z1 = _agg_kernel(y, src_r, dst_r)
        wn = W[l + 1] if l < LAYERS - 1 else W[0]
        h, y = _tc_post(l, z0, z1, y, h, dinv2d,
                        b[l].reshape(1, D), gamma[l].reshape(1, D),
                        beta[l].reshape(1, D), wn)
    return h


# speedup vs baseline: 2.3831x; 2.3831x over previous
import functools

import jax
import jax.numpy as jnp
from jax import lax
from jax.experimental import pallas as pl
from jax.experimental.pallas import tpu as pltpu
from jax.experimental.pallas import tpu_sc as plsc

N = 10000
E = 320000
D = 128
LAYERS = 4
EPS = 1e-05
LN_EPS = 1e-05

NC = 2
NS = 16
NW = NC * NS
CHUNK = 128
SB = 8
NSB = 10
EP = NW * NSB * SB * CHUNK
ACH = 80
ANC = 125
NPAD = 10240
RPT = NPAD // NS

_mesh = plsc.VectorSubcoreMesh(core_axis_name="c", subcore_axis_name="s")


@functools.partial(
    pl.kernel,
    out_type=(
        jax.ShapeDtypeStruct((NPAD,), jnp.float32),
        jax.ShapeDtypeStruct((NPAD,), jnp.float32),
    ),
    mesh=_mesh,
    scratch_types=[
        pltpu.VMEM((NSB, SB, CHUNK), jnp.int32),
        pltpu.VMEM((CHUNK,), jnp.float32),
        pltpu.VMEM_SHARED((NPAD,), jnp.float32),
    ],
)
def _deg_kernel(dst_hbm, zeros1d_hbm, ones_hbm, d0_hbm, d1_hbm,
                idxd_v, ones_v, deg_sp):
    cid = lax.axis_index("c")
    sid = lax.axis_index("s")
    w = cid * NS + sid
    pltpu.sync_copy(zeros1d_hbm, deg_sp.at[pl.ds(sid * RPT, RPT)])
    pltpu.sync_copy(ones_hbm, ones_v)
    pltpu.sync_copy(dst_hbm.at[w], idxd_v)
    plsc.subcore_barrier()

    @pl.loop(0, NSB)
    def _(sb):
        for k in range(SB):
            pltpu.sync_copy(ones_v, deg_sp.at[idxd_v.at[sb, k]], add=True)

    plsc.subcore_barrier()

    @pl.when(jnp.logical_and(sid == 0, cid == 0))
    def _():
        pltpu.sync_copy(deg_sp, d0_hbm)

    @pl.when(jnp.logical_and(sid == 0, cid == 1))
    def _():
        pltpu.sync_copy(deg_sp, d1_hbm)


@functools.partial(
    pl.kernel,
    out_type=(
        jax.ShapeDtypeStruct((NPAD, D), jnp.float32),
        jax.ShapeDtypeStruct((NPAD, D), jnp.float32),
    ),
    mesh=_mesh,
    scratch_types=[
        pltpu.VMEM((ANC, ACH), jnp.int32),
        pltpu.VMEM((ANC, ACH), jnp.int32),
        pltpu.VMEM((ACH, D), jnp.float32),
        pltpu.VMEM_SHARED((NPAD, D), jnp.float32),
        pltpu.SemaphoreType.DMA,
    ],
)
def _agg_kernel(y_hbm, src_hbm, dst_hbm, zeros2d_hbm, z0_hbm, z1_hbm,
                idxs_v, idxd_v, rows_v, z_sp, sem):
    cid = lax.axis_index("c")
    sid = lax.axis_index("s")
    w = cid * NS + sid

    pltpu.sync_copy(zeros2d_hbm, z_sp.at[pl.ds(sid * RPT, RPT)])
    pltpu.sync_copy(src_hbm.at[w], idxs_v)
    pltpu.sync_copy(dst_hbm.at[w], idxd_v)
    plsc.subcore_barrier()

    @pl.loop(0, ANC)
    def _(j):
        pltpu.async_copy(y_hbm.at[idxs_v.at[j]], rows_v, sem).wait()
        pltpu.sync_copy(rows_v, z_sp.at[idxd_v.at[j]], add=True)

    plsc.subcore_barrier()

    @pl.when(cid == 0)
    def _():
        pltpu.sync_copy(z_sp.at[pl.ds(sid * RPT, RPT)],
                        z0_hbm.at[pl.ds(sid * RPT, RPT)])

    @pl.when(cid == 1)
    def _():
        pltpu.sync_copy(z_sp.at[pl.ds(sid * RPT, RPT)],
                        z1_hbm.at[pl.ds(sid * RPT, RPT)])


def _prep_body(degs_ref, x_ref, w_ref, y_ref, dinv_ref):
    d = degs_ref[:, 0] + degs_ref[:, 1] + 1.0
    dinv = lax.rsqrt(d)[:, None]
    dinv_ref[...] = jnp.broadcast_to(dinv, x_ref.shape)
    y_ref[...] = dinv * jnp.dot(x_ref[...], w_ref[...],
                                preferred_element_type=jnp.float32)


def _post_body(layer, z0_ref, z1_ref, y_ref, xin_ref, dinv_ref,
               b_ref, g_ref, bt_ref, wn_ref, h_ref, yn_ref):
    dinv = dinv_ref[...]
    h = dinv * (z0_ref[...] + z1_ref[...] + y_ref[...]) + b_ref[...]
    h = jnp.where(jnp.isnan(h), jnp.float32(0.0), h)
    h = jnp.where(jnp.isinf(h) & (h > 0), jnp.float32(EPS), h)
    h = jnp.where(jnp.isinf(h) & (h < 0), jnp.float32(-EPS), h)
    mu = jnp.mean(h, axis=-1, keepdims=True)
    var = jnp.mean((h - mu) ** 2, axis=-1, keepdims=True)
    h = (h - mu) / jnp.sqrt(var + LN_EPS) * g_ref[...] + bt_ref[...]
    if layer > 0:
        h = h + xin_ref[...]
    if layer < LAYERS - 1:
        h = jax.nn.relu(h)
    h_ref[...] = h
    if layer < LAYERS - 1:
        yn_ref[...] = dinv * jnp.dot(h, wn_ref[...],
                                     preferred_element_type=jnp.float32)


_BN = 1000


def _tc_prep(degs, x, w0):
    return pl.pallas_call(
        _prep_body,
        grid=(N // _BN,),
        in_specs=[
            pl.BlockSpec((_BN, 2), lambda i: (i, 0)),
            pl.BlockSpec((_BN, D), lambda i: (i, 0)),
            pl.BlockSpec((D, D), lambda i: (0, 0)),
        ],
        out_specs=[
            pl.BlockSpec((_BN, D), lambda i: (i, 0)),
            pl.BlockSpec((_BN, D), lambda i: (i, 0)),
        ],
        out_shape=[
            jax.ShapeDtypeStruct((N, D), jnp.float32),
            jax.ShapeDtypeStruct((N, D), jnp.float32),
        ],
    )(degs, x, w0)


def _tc_post(layer, z0, z1, y, xin, dinv2d, bl, gl, btl, wn):
    last = layer == LAYERS - 1
    if last:
        def body(z0r, z1r, yr, xr, dr, br, gr, btr, wr, hr):
            _post_body(layer, z0r, z1r, yr, xr, dr, br, gr, btr, wr, hr, None)
        out_specs = [pl.BlockSpec((_BN, D), lambda i: (i, 0))]
        out_shape = [jax.ShapeDtypeStruct((N, D), jnp.float32)]
    else:
        body = functools.partial(_post_body, layer)
        out_specs = [pl.BlockSpec((_BN, D), lambda i: (i, 0))] * 2
        out_shape = [jax.ShapeDtypeStruct((N, D), jnp.float32)] * 2
    res = pl.pallas_call(
        body,
        grid=(N // _BN,),
        in_specs=[
            pl.BlockSpec((_BN, D), lambda i: (i, 0)),
            pl.BlockSpec((_BN, D), lambda i: (i, 0)),
            pl.BlockSpec((_BN, D), lambda i: (i, 0)),
            pl.BlockSpec((_BN, D), lambda i: (i, 0)),
            pl.BlockSpec((_BN, D), lambda i: (i, 0)),
            pl.BlockSpec((1, D), lambda i: (0, 0)),
            pl.BlockSpec((1, D), lambda i: (0, 0)),
            pl.BlockSpec((1, D), lambda i: (0, 0)),
            pl.BlockSpec((D, D), lambda i: (0, 0)),
        ],
        out_specs=out_specs,
        out_shape=out_shape,
    )(z0, z1, y, xin, dinv2d, bl, gl, btl, wn)
    return res if not last else (res[0], None)


@jax.jit
def kernel(x, edge_index, W, b, gamma, beta):
    npad_e = EP - E
    dst_r = jnp.concatenate(
        [edge_index[1], jnp.full((npad_e,), NPAD - 1, jnp.int32)]
    ).reshape(NW, NSB, SB, CHUNK)
    src_a = edge_index[0].reshape(NW, ANC, ACH)
    dst_a = edge_index[1].reshape(NW, ANC, ACH)
    zeros1d = jnp.zeros((RPT,), jnp.float32)
    zeros2d = jnp.zeros((RPT, D), jnp.float32)
    ones_c = jnp.ones((CHUNK,), jnp.float32)

    d0, d1 = _deg_kernel(dst_r, zeros1d, ones_c)
    degs = jnp.stack([d0[:N], d1[:N]], axis=1)
    y, dinv2d = _tc_prep(degs, x, W[0])

    h = x
    for l in range(LAYERS):
        z0, z1 = _agg_kernel(y, src_a, dst_a, zeros2d)
        wn = W[l + 1] if l < LAYERS - 1 else W[0]
        h, y = _tc_post(l, z0, z1, y, h, dinv2d,
                        b[l].reshape(1, D), gamma[l].reshape(1, D),
                        beta[l].reshape(1, D), wn)
    return h


# speedup vs baseline: 2.7420x; 1.1506x over previous
import functools

import jax
import jax.numpy as jnp
from jax import lax
from jax.experimental import pallas as pl
from jax.experimental.pallas import tpu as pltpu
from jax.experimental.pallas import tpu_sc as plsc

N = 10000
E = 320000
D = 128
LAYERS = 4
EPS = 1e-05
LN_EPS = 1e-05

NC = 2
NS = 16
NW = NC * NS
CHUNK = 128
SB = 8
NSB = 10
EP = NW * NSB * SB * CHUNK
ACH = 125
ANC = 80
NPAD = 10240
RPT = NPAD // NS

_mesh = plsc.VectorSubcoreMesh(core_axis_name="c", subcore_axis_name="s")


@functools.partial(
    pl.kernel,
    out_type=(
        jax.ShapeDtypeStruct((NPAD,), jnp.float32),
        jax.ShapeDtypeStruct((NPAD,), jnp.float32),
    ),
    mesh=_mesh,
    scratch_types=[
        pltpu.VMEM((NSB, SB, CHUNK), jnp.int32),
        pltpu.VMEM((CHUNK,), jnp.float32),
        pltpu.VMEM_SHARED((NPAD,), jnp.float32),
    ],
)
def _deg_kernel(dst_hbm, zeros1d_hbm, ones_hbm, d0_hbm, d1_hbm,
                idxd_v, ones_v, deg_sp):
    cid = lax.axis_index("c")
    sid = lax.axis_index("s")
    w = cid * NS + sid
    pltpu.sync_copy(zeros1d_hbm, deg_sp.at[pl.ds(sid * RPT, RPT)])
    pltpu.sync_copy(ones_hbm, ones_v)
    pltpu.sync_copy(dst_hbm.at[w], idxd_v)
    plsc.subcore_barrier()

    @pl.loop(0, NSB)
    def _(sb):
        for k in range(SB):
            pltpu.sync_copy(ones_v, deg_sp.at[idxd_v.at[sb, k]], add=True)

    plsc.subcore_barrier()

    @pl.when(jnp.logical_and(sid == 0, cid == 0))
    def _():
        pltpu.sync_copy(deg_sp, d0_hbm)

    @pl.when(jnp.logical_and(sid == 0, cid == 1))
    def _():
        pltpu.sync_copy(deg_sp, d1_hbm)


@functools.partial(
    pl.kernel,
    out_type=(
        jax.ShapeDtypeStruct((NPAD, D), jnp.float32),
        jax.ShapeDtypeStruct((NPAD, D), jnp.float32),
    ),
    mesh=_mesh,
    scratch_types=[
        pltpu.VMEM((ANC, ACH), jnp.int32),
        pltpu.VMEM((ANC, ACH), jnp.int32),
        pltpu.VMEM((ACH, D), jnp.float32),
        pltpu.VMEM_SHARED((NPAD, D), jnp.float32),
        pltpu.SemaphoreType.DMA,
    ],
)
def _agg_kernel(y_hbm, src_hbm, dst_hbm, zeros2d_hbm, z0_hbm, z1_hbm,
                idxs_v, idxd_v, rows_v, z_sp, sem):
    cid = lax.axis_index("c")
    sid = lax.axis_index("s")
    w = cid * NS + sid

    pltpu.sync_copy(zeros2d_hbm, z_sp.at[pl.ds(sid * RPT, RPT)])
    pltpu.sync_copy(src_hbm.at[w], idxs_v)
    pltpu.sync_copy(dst_hbm.at[w], idxd_v)
    plsc.subcore_barrier()

    @pl.loop(0, ANC)
    def _(j):
        pltpu.async_copy(y_hbm.at[idxs_v.at[j]], rows_v, sem).wait()
        pltpu.sync_copy(rows_v, z_sp.at[idxd_v.at[j]], add=True)

    plsc.subcore_barrier()

    @pl.when(cid == 0)
    def _():
        pltpu.sync_copy(z_sp.at[pl.ds(sid * RPT, RPT)],
                        z0_hbm.at[pl.ds(sid * RPT, RPT)])

    @pl.when(cid == 1)
    def _():
        pltpu.sync_copy(z_sp.at[pl.ds(sid * RPT, RPT)],
                        z1_hbm.at[pl.ds(sid * RPT, RPT)])


def _prep_body(degs_ref, x_ref, w_ref, y_ref, dinv_ref):
    d = degs_ref[:, 0] + degs_ref[:, 1] + 1.0
    dinv = lax.rsqrt(d)[:, None]
    dinv_ref[...] = jnp.broadcast_to(dinv, x_ref.shape)
    y_ref[...] = dinv * jnp.dot(x_ref[...], w_ref[...],
                                preferred_element_type=jnp.float32)


def _post_body(layer, z0_ref, z1_ref, y_ref, xin_ref, dinv_ref,
               b_ref, g_ref, bt_ref, wn_ref, h_ref, yn_ref):
    dinv = dinv_ref[...]
    h = dinv * (z0_ref[...] + z1_ref[...] + y_ref[...]) + b_ref[...]
    h = jnp.where(jnp.isnan(h), jnp.float32(0.0), h)
    h = jnp.where(jnp.isinf(h) & (h > 0), jnp.float32(EPS), h)
    h = jnp.where(jnp.isinf(h) & (h < 0), jnp.float32(-EPS), h)
    mu = jnp.mean(h, axis=-1, keepdims=True)
    var = jnp.mean((h - mu) ** 2, axis=-1, keepdims=True)
    h = (h - mu) / jnp.sqrt(var + LN_EPS) * g_ref[...] + bt_ref[...]
    if layer > 0:
        h = h + xin_ref[...]
    if layer < LAYERS - 1:
        h = jax.nn.relu(h)
    h_ref[...] = h
    if layer < LAYERS - 1:
        yn_ref[...] = dinv * jnp.dot(h, wn_ref[...],
                                     preferred_element_type=jnp.float32)


_BN = 1000


def _tc_prep(degs, x, w0):
    return pl.pallas_call(
        _prep_body,
        grid=(N // _BN,),
        in_specs=[
            pl.BlockSpec((_BN, 2), lambda i: (i, 0)),
            pl.BlockSpec((_BN, D), lambda i: (i, 0)),
            pl.BlockSpec((D, D), lambda i: (0, 0)),
        ],
        out_specs=[
            pl.BlockSpec((_BN, D), lambda i: (i, 0)),
            pl.BlockSpec((_BN, D), lambda i: (i, 0)),
        ],
        out_shape=[
            jax.ShapeDtypeStruct((N, D), jnp.float32),
            jax.ShapeDtypeStruct((N, D), jnp.float32),
        ],
    )(degs, x, w0)


def _tc_post(layer, z0, z1, y, xin, dinv2d, bl, gl, btl, wn):
    last = layer == LAYERS - 1
    if last:
        def body(z0r, z1r, yr, xr, dr, br, gr, btr, wr, hr):
            _post_body(layer, z0r, z1r, yr, xr, dr, br, gr, btr, wr, hr, None)
        out_specs = [pl.BlockSpec((_BN, D), lambda i: (i, 0))]
        out_shape = [jax.ShapeDtypeStruct((N, D), jnp.float32)]
    else:
        body = functools.partial(_post_body, layer)
        out_specs = [pl.BlockSpec((_BN, D), lambda i: (i, 0))] * 2
        out_shape = [jax.ShapeDtypeStruct((N, D), jnp.float32)] * 2
    res = pl.pallas_call(
        body,
        grid=(N // _BN,),
        in_specs=[
            pl.BlockSpec((_BN, D), lambda i: (i, 0)),
            pl.BlockSpec((_BN, D), lambda i: (i, 0)),
            pl.BlockSpec((_BN, D), lambda i: (i, 0)),
            pl.BlockSpec((_BN, D), lambda i: (i, 0)),
            pl.BlockSpec((_BN, D), lambda i: (i, 0)),
            pl.BlockSpec((1, D), lambda i: (0, 0)),
            pl.BlockSpec((1, D), lambda i: (0, 0)),
            pl.BlockSpec((1, D), lambda i: (0, 0)),
            pl.BlockSpec((D, D), lambda i: (0, 0)),
        ],
        out_specs=out_specs,
        out_shape=out_shape,
    )(z0, z1, y, xin, dinv2d, bl, gl, btl, wn)
    return res if not last else (res[0], None)


@jax.jit
def kernel(x, edge_index, W, b, gamma, beta):
    npad_e = EP - E
    dst_r = jnp.concatenate(
        [edge_index[1], jnp.full((npad_e,), NPAD - 1, jnp.int32)]
    ).reshape(NW, NSB, SB, CHUNK)
    src_a = edge_index[0].reshape(NW, ANC, ACH)
    dst_a = edge_index[1].reshape(NW, ANC, ACH)
    zeros1d = jnp.zeros((RPT,), jnp.float32)
    zeros2d = jnp.zeros((RPT, D), jnp.float32)
    ones_c = jnp.ones((CHUNK,), jnp.float32)

    d0, d1 = _deg_kernel(dst_r, zeros1d, ones_c)
    degs = jnp.stack([d0[:N], d1[:N]], axis=1)
    y, dinv2d = _tc_prep(degs, x, W[0])

    h = x
    for l in range(LAYERS):
        z0, z1 = _agg_kernel(y, src_a, dst_a, zeros2d)
        wn = W[l + 1] if l < LAYERS - 1 else W[0]
        h, y = _tc_post(l, z0, z1, y, h, dinv2d,
                        b[l].reshape(1, D), gamma[l].reshape(1, D),
                        beta[l].reshape(1, D), wn)
    return h
